# baseline (speedup 1.0000x reference)
import functools

import jax
import jax.numpy as jnp
from jax import lax
from jax.experimental import pallas as pl
from jax.experimental.pallas import tpu as pltpu
from jax.experimental.pallas import tpu_sc as plsc

B = 4096
V = 1000000
D = 16
F = 26
L = 20
DOM = 10
TF = F * D
TFP = 512
ATT_HID = 128
ATT_OUT = 64
FIN_HID = 64
FL = F * L

NC = 2
NS = 16
NW = NC * NS
B_PER_W = B // NW
CH = 8
NCHUNK = B_PER_W // CH
QF = TFP // 128
OUT_ROWS = B * QF


def _gather_sum(ids2d, table):
    mesh = plsc.VectorSubcoreMesh(core_axis_name="c", subcore_axis_name="s")

    @functools.partial(
        pl.kernel,
        out_type=jax.ShapeDtypeStruct((OUT_ROWS, 128), jnp.float32),
        mesh=mesh,
        scratch_types=[
            pltpu.VMEM((CH, FL), jnp.int32),
            pltpu.VMEM((CH * FL, D), jnp.float32),
            pltpu.VMEM((CH * QF, 128), jnp.float32),
            pltpu.SemaphoreType.DMA,
        ],
        compiler_params=pltpu.CompilerParams(use_tc_tiling_on_sc=False),
    )
    def k(ids_hbm, table_hbm, out_hbm, idx_v, rows_v, acc_v, sem):
        wid = lax.axis_index("s") * NC + lax.axis_index("c")
        zero16 = jnp.zeros((D,), jnp.float32)

        for j in range(CH):
            for c0 in range(TF % 128, 128, D):
                acc_v[j * QF + QF - 1, pl.ds(c0, D)] = zero16

        @pl.loop(0, NCHUNK)
        def _(c):
            r0 = wid * B_PER_W + c * CH
            pltpu.sync_copy(ids_hbm.at[pl.ds(r0, CH)], idx_v)
            copies = []
            for j in range(CH):
                copies.append(pltpu.async_copy(
                    table_hbm.at[idx_v.at[j]],
                    rows_v.at[pl.ds(j * FL, FL)], sem))
            for cp in copies:
                cp.wait()

            for j in range(CH):
                for f in range(F):
                    acc = rows_v[j * FL + f * L]
                    for l in range(1, L):
                        acc = acc + rows_v[j * FL + f * L + l]
                    acc_v[j * QF + f // 8, pl.ds((f % 8) * D, D)] = acc

            pltpu.sync_copy(acc_v, out_hbm.at[pl.ds(r0 * QF, CH * QF)])

    return k(ids2d, table)


BB = 512


def _tail_body(emb_ref, did_ref, dt_ref, w1a_ref, w1d_ref, b1_ref, w2_ref,
               b2_ref, wo_ref, bo_ref, wf1a_ref, wf1d_ref, bf1_ref, wf2_ref,
               bf2_ref, out_ref):
    e = emb_ref[...].reshape(BB, TFP)
    did = did_ref[...]
    oh = (did == lax.broadcasted_iota(jnp.int32, (BB, DOM), 1))
    de = jnp.dot(oh.astype(jnp.float32), dt_ref[...],
                 preferred_element_type=jnp.float32)
    h = jnp.dot(e, w1a_ref[...], preferred_element_type=jnp.float32)
    h = h + jnp.dot(de, w1d_ref[...], preferred_element_type=jnp.float32)
    h = jnp.maximum(h + b1_ref[...], 0.0)
    h = jnp.maximum(
        jnp.dot(h, w2_ref[...], preferred_element_type=jnp.float32)
        + b2_ref[...], 0.0)
    aw = jnp.dot(h, wo_ref[...], preferred_element_type=jnp.float32)
    aw = aw + bo_ref[...]
    aw = aw - jnp.max(aw, axis=1, keepdims=True)
    ex = jnp.exp(aw)
    aw = ex / jnp.sum(ex, axis=1, keepdims=True)
    w = e * aw
    hh = jnp.dot(w, wf1a_ref[...], preferred_element_type=jnp.float32)
    hh = hh + jnp.dot(de, wf1d_ref[...], preferred_element_type=jnp.float32)
    hh = jnp.maximum(hh + bf1_ref[...], 0.0)
    logit = jnp.dot(hh, wf2_ref[...], preferred_element_type=jnp.float32)
    logit = logit + bf2_ref[...]
    out_ref[...] = 1.0 / (1.0 + jnp.exp(-logit))


def _tail(emb, did, dom_table, w1a, w1d, b1, w2, b2, wo, bo, wf1a, wf1d,
          bf1, wf2, bf2):
    full = lambda shape: pl.BlockSpec(shape, lambda i: (0, 0))
    return pl.pallas_call(
        _tail_body,
        grid=(B // BB,),
        in_specs=[
            pl.BlockSpec((BB * QF, 128), lambda i: (i, 0)),
            pl.BlockSpec((BB, 1), lambda i: (i, 0)),
            full((DOM, D)),
            full((TFP, ATT_HID)),
            full((D, ATT_HID)),
            full((1, ATT_HID)),
            full((ATT_HID, ATT_OUT)),
            full((1, ATT_OUT)),
            full((ATT_OUT, TFP)),
            full((1, TFP)),
            full((TFP, FIN_HID)),
            full((D, FIN_HID)),
            full((1, FIN_HID)),
            full((FIN_HID, 1)),
            full((1, 1)),
        ],
        out_specs=pl.BlockSpec((BB, 1), lambda i: (i, 0)),
        out_shape=jax.ShapeDtypeStruct((B, 1), jnp.float32),
    )(emb, did, dom_table, w1a, w1d, b1, w2, b2, wo, bo, wf1a, wf1d, bf1,
      wf2, bf2)


def kernel(x, domain_ids, table, dom_table, W1, b1, W2, b2, Wo, bo, Wf1,
           bf1, Wf2, bf2):
    ids2d = x.reshape(B, FL)
    emb = _gather_sum(ids2d, table)
    npad = TFP - TF
    w1a = jnp.concatenate([W1[:TF], jnp.zeros((npad, ATT_HID), W1.dtype)])
    w1d = W1[TF:].reshape(F, D, ATT_HID).sum(axis=0)
    wo_p = jnp.concatenate([Wo, jnp.zeros((ATT_OUT, npad), Wo.dtype)], axis=1)
    bo_p = jnp.concatenate([bo, jnp.full((npad,), -1e30, bo.dtype)])
    wf1a = jnp.concatenate([Wf1[:TF], jnp.zeros((npad, FIN_HID), Wf1.dtype)])
    wf1d = Wf1[TF:]
    did = domain_ids.reshape(B, 1).astype(jnp.int32)
    return _tail(emb, did, dom_table, w1a, w1d, b1.reshape(1, -1), W2,
                 b2.reshape(1, -1), wo_p, bo_p.reshape(1, -1), wf1a, wf1d,
                 bf1.reshape(1, -1), Wf2, bf2.reshape(1, 1))


# speedup vs baseline: 1.5336x; 1.5336x over previous
import functools

import jax
import jax.numpy as jnp
from jax import lax
from jax.experimental import pallas as pl
from jax.experimental.pallas import tpu as pltpu
from jax.experimental.pallas import tpu_sc as plsc

B = 4096
V = 1000000
D = 16
F = 26
L = 20
DOM = 10
TF = F * D
TFP = 512
ATT_HID = 128
ATT_OUT = 64
FIN_HID = 64

NC = 2
NS = 16
NW = NC * NS
B_PER_W = B // NW
QF = TFP // 128
NPADL = 128 - TF % 128

CW = 8192
NBLK = -(-V // CW)
V2 = NBLK * CW


def _detile_body(tt_ref, out_ref):
    x = tt_ref[...]
    t = x.T.reshape(CW // 8, 8, D)
    out_ref[...] = jnp.concatenate(
        [t[:, p, :] for p in range(8)], axis=1)


def _detile(tt):
    return pl.pallas_call(
        _detile_body,
        grid=(NBLK,),
        in_specs=[pl.BlockSpec((D, CW), lambda i: (0, i))],
        out_specs=pl.BlockSpec((CW // 8, 128), lambda i: (i, 0)),
        out_shape=jax.ShapeDtypeStruct((V2 // 8, 128), jnp.float32),
    )(tt)


def _gather_sum(xt, table):
    mesh = plsc.VectorSubcoreMesh(core_axis_name="c", subcore_axis_name="s")

    @functools.partial(
        pl.kernel,
        out_type=jax.ShapeDtypeStruct((B, QF, 128), jnp.float32),
        mesh=mesh,
        scratch_types=[
            pltpu.VMEM((L, B_PER_W), jnp.int32),
            pltpu.VMEM((L * B_PER_W, D), jnp.float32),
            pltpu.VMEM((B_PER_W, D), jnp.float32),
            pltpu.VMEM((B_PER_W, NPADL), jnp.float32),
            pltpu.SemaphoreType.DMA,
        ],
        compiler_params=pltpu.CompilerParams(use_tc_tiling_on_sc=False),
    )
    def k(xt_hbm, table_hbm, out_hbm, idx_v, rows_v, acc_v, zer_v, sem):
        wid = lax.axis_index("s") * NC + lax.axis_index("c")
        b0 = wid * B_PER_W
        zero16 = jnp.zeros((D,), jnp.float32)

        @pl.loop(0, B_PER_W)
        def _(j):
            for c in range(NPADL // D):
                zer_v[j, pl.ds(c * D, D)] = zero16
        pltpu.sync_copy(
            zer_v, out_hbm.at[pl.ds(b0, B_PER_W), QF - 1, pl.ds(TF % 128, NPADL)])

        @pl.loop(0, F)
        def _(f):
            pltpu.sync_copy(xt_hbm.at[f, :, pl.ds(b0, B_PER_W)], idx_v)
            copies = []
            for l in range(L):
                copies.append(pltpu.async_copy(
                    table_hbm.at[idx_v.at[l]],
                    rows_v.at[pl.ds(l * B_PER_W, B_PER_W)], sem))
            for cp in copies:
                cp.wait()

            @pl.loop(0, B_PER_W)
            def _(j):
                acc = rows_v[j]
                for l in range(1, L):
                    acc = acc + rows_v[l * B_PER_W + j]
                acc_v[j] = acc

            pltpu.sync_copy(
                acc_v,
                out_hbm.at[pl.ds(b0, B_PER_W), f // 8, pl.ds((f % 8) * D, D)])

    return k(xt, table)


BB = 512


def _tail_body(emb_ref, did_ref, dt_ref, w1a_ref, w1d_ref, b1_ref, w2_ref,
               b2_ref, wo_ref, bo_ref, wf1a_ref, wf1d_ref, bf1_ref, wf2_ref,
               bf2_ref, out_ref):
    e = emb_ref[...].reshape(BB, TFP)
    did = did_ref[...]
    oh = (did == lax.broadcasted_iota(jnp.int32, (BB, DOM), 1))
    de = jnp.dot(oh.astype(jnp.float32), dt_ref[...],
                 preferred_element_type=jnp.float32)
    h = jnp.dot(e, w1a_ref[...], preferred_element_type=jnp.float32)
    h = h + jnp.dot(de, w1d_ref[...], preferred_element_type=jnp.float32)
    h = jnp.maximum(h + b1_ref[...], 0.0)
    h = jnp.maximum(
        jnp.dot(h, w2_ref[...], preferred_element_type=jnp.float32)
        + b2_ref[...], 0.0)
    aw = jnp.dot(h, wo_ref[...], preferred_element_type=jnp.float32)
    aw = aw + bo_ref[...]
    aw = aw - jnp.max(aw, axis=1, keepdims=True)
    ex = jnp.exp(aw)
    aw = ex / jnp.sum(ex, axis=1, keepdims=True)
    w = e * aw
    hh = jnp.dot(w, wf1a_ref[...], preferred_element_type=jnp.float32)
    hh = hh + jnp.dot(de, wf1d_ref[...], preferred_element_type=jnp.float32)
    hh = jnp.maximum(hh + bf1_ref[...], 0.0)
    logit = jnp.dot(hh, wf2_ref[...], preferred_element_type=jnp.float32)
    logit = logit + bf2_ref[...]
    out_ref[...] = 1.0 / (1.0 + jnp.exp(-logit))


def _tail(emb, did, dom_table, w1a, w1d, b1, w2, b2, wo, bo, wf1a, wf1d,
          bf1, wf2, bf2):
    full = lambda shape: pl.BlockSpec(shape, lambda i: (0, 0))
    return pl.pallas_call(
        _tail_body,
        grid=(B // BB,),
        in_specs=[
            pl.BlockSpec((BB, QF, 128), lambda i: (i, 0, 0)),
            pl.BlockSpec((BB, 1), lambda i: (i, 0)),
            full((DOM, D)),
            full((TFP, ATT_HID)),
            full((D, ATT_HID)),
            full((1, ATT_HID)),
            full((ATT_HID, ATT_OUT)),
            full((1, ATT_OUT)),
            full((ATT_OUT, TFP)),
            full((1, TFP)),
            full((TFP, FIN_HID)),
            full((D, FIN_HID)),
            full((1, FIN_HID)),
            full((FIN_HID, 1)),
            full((1, 1)),
        ],
        out_specs=pl.BlockSpec((BB, 1), lambda i: (i, 0)),
        out_shape=jax.ShapeDtypeStruct((B, 1), jnp.float32),
    )(emb, did, dom_table, w1a, w1d, b1, w2, b2, wo, bo, wf1a, wf1d, bf1,
      wf2, bf2)


def kernel(x, domain_ids, table, dom_table, W1, b1, W2, b2, Wo, bo, Wf1,
           bf1, Wf2, bf2):
    tt = table.T
    tl = _detile(tt)
    tlv = tl.reshape(V2, D)
    xt = jnp.transpose(x, (1, 2, 0))
    emb = _gather_sum(xt, tlv)
    npad = TFP - TF
    w1a = jnp.concatenate([W1[:TF], jnp.zeros((npad, ATT_HID), W1.dtype)])
    w1d = W1[TF:].reshape(F, D, ATT_HID).sum(axis=0)
    wo_p = jnp.concatenate([Wo, jnp.zeros((ATT_OUT, npad), Wo.dtype)], axis=1)
    bo_p = jnp.concatenate([bo, jnp.full((npad,), -1e30, bo.dtype)])
    wf1a = jnp.concatenate([Wf1[:TF], jnp.zeros((npad, FIN_HID), Wf1.dtype)])
    wf1d = Wf1[TF:]
    did = domain_ids.reshape(B, 1).astype(jnp.int32)
    return _tail(emb, did, dom_table, w1a, w1d, b1.reshape(1, -1), W2,
                 b2.reshape(1, -1), wo_p, bo_p.reshape(1, -1), wf1a, wf1d,
                 bf1.reshape(1, -1), Wf2, bf2.reshape(1, 1))


# speedup vs baseline: 1.6728x; 1.0908x over previous
import functools

import jax
import jax.numpy as jnp
from jax import lax
from jax.experimental import pallas as pl
from jax.experimental.pallas import tpu as pltpu
from jax.experimental.pallas import tpu_sc as plsc

B = 4096
V = 1000000
D = 16
F = 26
L = 20
DOM = 10
TF = F * D
TFP = 512
ATT_HID = 128
ATT_OUT = 64
FIN_HID = 64

NC = 2
NS = 16
NW = NC * NS
B_PER_W = B // NW
QF = TFP // 128
NPADL = 128 - TF % 128

CW = 8192
NBLK = -(-V // CW)
V2 = NBLK * CW


def _detile_body(tt_ref, out_ref):
    x = tt_ref[...]
    t = x.T.reshape(CW // 8, 8, D)
    for p in range(8):
        out_ref[:, pl.ds(p * D, D)] = t[:, p, :]


def _detile(tt):
    return pl.pallas_call(
        _detile_body,
        grid=(NBLK,),
        in_specs=[pl.BlockSpec((D, CW), lambda i: (0, i))],
        out_specs=pl.BlockSpec((CW // 8, 128), lambda i: (i, 0)),
        out_shape=jax.ShapeDtypeStruct((V2 // 8, 128), jnp.float32),
    )(tt)


def _gather_sum(xt, table):
    mesh = plsc.VectorSubcoreMesh(core_axis_name="c", subcore_axis_name="s")

    @functools.partial(
        pl.kernel,
        out_type=jax.ShapeDtypeStruct((B, QF, 128), jnp.float32),
        mesh=mesh,
        scratch_types=[
            pltpu.VMEM((L, B_PER_W), jnp.int32),
            pltpu.VMEM((L * B_PER_W, D), jnp.float32),
            pltpu.VMEM((B_PER_W, D), jnp.float32),
            pltpu.VMEM((B_PER_W, NPADL), jnp.float32),
            pltpu.SemaphoreType.DMA,
        ],
        compiler_params=pltpu.CompilerParams(use_tc_tiling_on_sc=False),
    )
    def k(xt_hbm, table_hbm, out_hbm, idx_v, rows_v, acc_v, zer_v, sem):
        wid = lax.axis_index("s") * NC + lax.axis_index("c")
        b0 = wid * B_PER_W
        zero16 = jnp.zeros((D,), jnp.float32)

        @pl.loop(0, B_PER_W)
        def _(j):
            for c in range(NPADL // D):
                zer_v[j, pl.ds(c * D, D)] = zero16
        pltpu.sync_copy(
            zer_v, out_hbm.at[pl.ds(b0, B_PER_W), QF - 1, pl.ds(TF % 128, NPADL)])

        @pl.loop(0, F)
        def _(f):
            pltpu.sync_copy(xt_hbm.at[f, :, pl.ds(b0, B_PER_W)], idx_v)
            copies = []
            for l in range(L):
                copies.append(pltpu.async_copy(
                    table_hbm.at[idx_v.at[l]],
                    rows_v.at[pl.ds(l * B_PER_W, B_PER_W)], sem))
            for cp in copies:
                cp.wait()

            @pl.loop(0, B_PER_W)
            def _(j):
                acc = rows_v[j]
                for l in range(1, L):
                    acc = acc + rows_v[l * B_PER_W + j]
                acc_v[j] = acc

            pltpu.sync_copy(
                acc_v,
                out_hbm.at[pl.ds(b0, B_PER_W), f // 8, pl.ds((f % 8) * D, D)])

    return k(xt, table)


BB = 512


def _tail_body(emb_ref, did_ref, dt_ref, w1a_ref, w1d_ref, b1_ref, w2_ref,
               b2_ref, wo_ref, bo_ref, wf1a_ref, wf1d_ref, bf1_ref, wf2_ref,
               bf2_ref, out_ref):
    e = emb_ref[...].reshape(BB, TFP)
    did = did_ref[...]
    oh = (did == lax.broadcasted_iota(jnp.int32, (BB, DOM), 1))
    de = jnp.dot(oh.astype(jnp.float32), dt_ref[...],
                 preferred_element_type=jnp.float32)
    h = jnp.dot(e, w1a_ref[...], preferred_element_type=jnp.float32)
    h = h + jnp.dot(de, w1d_ref[...], preferred_element_type=jnp.float32)
    h = jnp.maximum(h + b1_ref[...], 0.0)
    h = jnp.maximum(
        jnp.dot(h, w2_ref[...], preferred_element_type=jnp.float32)
        + b2_ref[...], 0.0)
    aw = jnp.dot(h, wo_ref[...], preferred_element_type=jnp.float32)
    aw = aw + bo_ref[...]
    aw = aw - jnp.max(aw, axis=1, keepdims=True)
    ex = jnp.exp(aw)
    aw = ex / jnp.sum(ex, axis=1, keepdims=True)
    w = e * aw
    hh = jnp.dot(w, wf1a_ref[...], preferred_element_type=jnp.float32)
    hh = hh + jnp.dot(de, wf1d_ref[...], preferred_element_type=jnp.float32)
    hh = jnp.maximum(hh + bf1_ref[...], 0.0)
    logit = jnp.dot(hh, wf2_ref[...], preferred_element_type=jnp.float32)
    logit = logit + bf2_ref[...]
    out_ref[...] = 1.0 / (1.0 + jnp.exp(-logit))


def _tail(emb, did, dom_table, w1a, w1d, b1, w2, b2, wo, bo, wf1a, wf1d,
          bf1, wf2, bf2):
    full = lambda shape: pl.BlockSpec(shape, lambda i: (0, 0))
    return pl.pallas_call(
        _tail_body,
        grid=(B // BB,),
        in_specs=[
            pl.BlockSpec((BB, QF, 128), lambda i: (i, 0, 0)),
            pl.BlockSpec((BB, 1), lambda i: (i, 0)),
            full((DOM, D)),
            full((TFP, ATT_HID)),
            full((D, ATT_HID)),
            full((1, ATT_HID)),
            full((ATT_HID, ATT_OUT)),
            full((1, ATT_OUT)),
            full((ATT_OUT, TFP)),
            full((1, TFP)),
            full((TFP, FIN_HID)),
            full((D, FIN_HID)),
            full((1, FIN_HID)),
            full((FIN_HID, 1)),
            full((1, 1)),
        ],
        out_specs=pl.BlockSpec((BB, 1), lambda i: (i, 0)),
        out_shape=jax.ShapeDtypeStruct((B, 1), jnp.float32),
    )(emb, did, dom_table, w1a, w1d, b1, w2, b2, wo, bo, wf1a, wf1d, bf1,
      wf2, bf2)


def kernel(x, domain_ids, table, dom_table, W1, b1, W2, b2, Wo, bo, Wf1,
           bf1, Wf2, bf2):
    tt = table.T
    tl = _detile(tt)
    tlv = tl.reshape(V2, D)
    xt = jnp.transpose(x, (1, 2, 0))
    emb = _gather_sum(xt, tlv)
    npad = TFP - TF
    w1a = jnp.concatenate([W1[:TF], jnp.zeros((npad, ATT_HID), W1.dtype)])
    w1d = W1[TF:].reshape(F, D, ATT_HID).sum(axis=0)
    wo_p = jnp.concatenate([Wo, jnp.zeros((ATT_OUT, npad), Wo.dtype)], axis=1)
    bo_p = jnp.concatenate([bo, jnp.full((npad,), -1e30, bo.dtype)])
    wf1a = jnp.concatenate([Wf1[:TF], jnp.zeros((npad, FIN_HID), Wf1.dtype)])
    wf1d = Wf1[TF:]
    did = domain_ids.reshape(B, 1).astype(jnp.int32)
    return _tail(emb, did, dom_table, w1a, w1d, b1.reshape(1, -1), W2,
                 b2.reshape(1, -1), wo_p, bo_p.reshape(1, -1), wf1a, wf1d,
                 bf1.reshape(1, -1), Wf2, bf2.reshape(1, 1))
